# Initial kernel scaffold; baseline (speedup 1.0000x reference)
import jax
import jax.numpy as jnp
from jax.experimental import pallas as pl


def kernel(x, edge_index, edge_weight, W0, b0, W1, b1, Wq, bq, Wk, bk, Wv, bv, Wo, bo, W2, b2, W3, b3):
    raise NotImplementedError("write your pallas kernel here")


# speedup vs baseline: 3.4162x; 3.4162x over previous
import functools

import jax
import jax.numpy as jnp
from jax import lax
from jax.experimental import pallas as pl
from jax.experimental.pallas import tpu as pltpu
from jax.experimental.pallas import tpu_sc as plsc

N = 10000
E = 320000
NFEAT = 128
H = 64
DM = 192
NCLASS = 40

NC = 2
NS = 16
CHUNK = 128
EPT = -(-E // NS)
NCH = -(-EPT // CHUNK)
EPAD = NS * NCH * CHUNK

BLK = 512
NP = 10240
NB = NP // BLK
KV = 1024
NKV = NP // KV
NPS = 10112
ROWS_PT = NPS // NS


def _sc_body(hcat, srcr, dstr, wr, out1, out2,
             accum, src_v, dst_v, w_v, rows, sem):
    c = lax.axis_index("c")
    s = lax.axis_index("s")
    roff = c * NPS
    row0 = s * ROWS_PT

    pltpu.sync_copy(srcr.at[s], src_v)
    pltpu.sync_copy(dstr.at[s], dst_v)
    pltpu.sync_copy(wr.at[s], w_v)

    def _add_off(j, _):
        for k in range(CHUNK // 16):
            sl = (j, pl.ds(k * 16, 16))
            src_v[sl] = src_v[sl] + roff
        return 0
    lax.fori_loop(0, NCH, _add_off, 0)

    pltpu.sync_copy(hcat.at[pl.ds(roff + row0, ROWS_PT)],
                    accum.at[pl.ds(row0, ROWS_PT)])
    plsc.subcore_barrier()

    def _edge_pass(h_hbm):
        def chunk_body(j, _):
            pltpu.async_copy(h_hbm.at[src_v.at[j]], rows, sem).wait()

            def wbody(i16, _):
                wvec = w_v[j, pl.ds(i16 * 16, 16)]
                for e in range(16):
                    wv = wvec[e]
                    i = i16 * 16 + e
                    for k in range(H // 16):
                        sl = (i, pl.ds(k * 16, 16))
                        rows[sl] = rows[sl] * wv
                return 0
            lax.fori_loop(0, CHUNK // 16, wbody, 0)
            pltpu.sync_copy(rows, accum.at[dst_v.at[j]], add=True)
            return 0
        lax.fori_loop(0, NCH, chunk_body, 0)

    _edge_pass(hcat)
    plsc.subcore_barrier()
    pltpu.sync_copy(accum.at[pl.ds(row0, ROWS_PT)],
                    out1.at[pl.ds(roff + row0, ROWS_PT)])
    plsc.subcore_barrier()

    _edge_pass(out1)
    plsc.subcore_barrier()
    pltpu.sync_copy(accum.at[pl.ds(row0, ROWS_PT)],
                    out2.at[pl.ds(roff + row0, ROWS_PT)])


def _sc_spmm(hflat, srcr, dstr, wr):
    fn = pl.kernel(
        _sc_body,
        out_type=(jax.ShapeDtypeStruct((NC * NPS, H), jnp.float32),
                  jax.ShapeDtypeStruct((NC * NPS, H), jnp.float32)),
        mesh=plsc.VectorSubcoreMesh(core_axis_name="c", subcore_axis_name="s",
                                    num_cores=NC, num_subcores=NS),
        compiler_params=pltpu.CompilerParams(use_tc_tiling_on_sc=False),
        scratch_types=[
            pltpu.VMEM_SHARED((NPS, H), jnp.float32),
            pltpu.VMEM((NCH, CHUNK), jnp.int32),
            pltpu.VMEM((NCH, CHUNK), jnp.int32),
            pltpu.VMEM((NCH, CHUNK), jnp.float32),
            pltpu.VMEM((CHUNK, H), jnp.float32),
            pltpu.SemaphoreType.DMA,
        ],
    )
    return fn(hflat, srcr, dstr, wr)


def _proj_body(x_ref, w0_ref, b0_ref, w1_ref, b1_ref, o_ref):
    xb = x_ref[...]
    o_ref[0] = jnp.maximum(xb @ w0_ref[...] + b0_ref[...], 0.0)
    o_ref[1] = jnp.maximum(xb @ w1_ref[...] + b1_ref[...], 0.0)


def _proj(x, W0, b0, W1, b1):
    return pl.pallas_call(
        _proj_body,
        grid=(NB,),
        in_specs=[
            pl.BlockSpec((BLK, NFEAT), lambda i: (i, 0)),
            pl.BlockSpec((NFEAT, H), lambda i: (0, 0)),
            pl.BlockSpec((1, H), lambda i: (0, 0)),
            pl.BlockSpec((NFEAT, H), lambda i: (0, 0)),
            pl.BlockSpec((1, H), lambda i: (0, 0)),
        ],
        out_specs=pl.BlockSpec((2, BLK, H), lambda i: (0, i, 0)),
        out_shape=jax.ShapeDtypeStruct((2, NPS, H), jnp.float32),
    )(x, W0, b0.reshape(1, H), W1, b1.reshape(1, H))


def _qkv_body(a_ref, b_ref, wq_ref, bq_ref, wk_ref, bk_ref, wv_ref, bv_ref,
              q_ref, k_ref, v_ref):
    i = pl.program_id(0)
    z1 = a_ref[0]
    z2 = b_ref[0]
    z3 = b_ref[1]
    rid = i * BLK + lax.broadcasted_iota(jnp.int32, (BLK, 1), 0)
    msk = rid < N

    def lin(w_ref, b_ref2):
        w = w_ref[...]
        acc = z1 @ w[0:H] + z2 @ w[H:2 * H] + z3 @ w[2 * H:3 * H]
        return jnp.where(msk, acc + b_ref2[...], 0.0)

    q_ref[...] = lin(wq_ref, bq_ref)
    k_ref[...] = lin(wk_ref, bk_ref)
    v_ref[...] = lin(wv_ref, bv_ref)


def _qkv(o1, o2, Wq, bq, Wk, bk, Wv, bv):
    wspec = pl.BlockSpec((DM, DM), lambda i: (0, 0))
    bspec = pl.BlockSpec((1, DM), lambda i: (0, 0))
    zspec = pl.BlockSpec((2, BLK, H), lambda i: (0, i, 0))
    ospec = pl.BlockSpec((BLK, DM), lambda i: (i, 0))
    oshape = jax.ShapeDtypeStruct((NP, DM), jnp.float32)
    return pl.pallas_call(
        _qkv_body,
        grid=(NB,),
        in_specs=[zspec, zspec, wspec, bspec, wspec, bspec, wspec, bspec],
        out_specs=(ospec, ospec, ospec),
        out_shape=(oshape, oshape, oshape),
    )(o1, o2, Wq, bq.reshape(1, DM), Wk, bk.reshape(1, DM),
      Wv, bv.reshape(1, DM))


def _flash_body(q_ref, k_ref, v_ref, o_ref):
    qb = q_ref[...] * (1.0 / (DM ** 0.5))

    def body(j, carry):
        m, l, acc = carry
        kb = k_ref[pl.ds(j * KV, KV), :]
        s = lax.dot_general(qb, kb, (((1,), (1,)), ((), ())))
        colid = j * KV + lax.broadcasted_iota(jnp.int32, (BLK, KV), 1)
        s = jnp.where(colid < N, s, -1e30)
        mnew = jnp.maximum(m, jnp.max(s, axis=1, keepdims=True))
        p = jnp.exp(s - mnew)
        scale = jnp.exp(m - mnew)
        lnew = l * scale + jnp.sum(p, axis=1, keepdims=True)
        vb = v_ref[pl.ds(j * KV, KV), :]
        accnew = acc * scale + lax.dot_general(p, vb, (((1,), (0,)), ((), ())))
        return mnew, lnew, accnew

    m0 = jnp.full((BLK, 1), -jnp.inf, jnp.float32)
    l0 = jnp.zeros((BLK, 1), jnp.float32)
    a0 = jnp.zeros((BLK, DM), jnp.float32)
    m, l, acc = lax.fori_loop(0, NKV, body, (m0, l0, a0))
    o_ref[...] = acc / l


def _flash(q, k, v):
    return pl.pallas_call(
        _flash_body,
        grid=(NB,),
        in_specs=[
            pl.BlockSpec((BLK, DM), lambda i: (i, 0)),
            pl.BlockSpec((NP, DM), lambda i: (0, 0)),
            pl.BlockSpec((NP, DM), lambda i: (0, 0)),
        ],
        out_specs=pl.BlockSpec((BLK, DM), lambda i: (i, 0)),
        out_shape=jax.ShapeDtypeStruct((NP, DM), jnp.float32),
    )(q, k, v)


def _final_body(a_ref, wo_ref, bo_ref, w2_ref, b2_ref, w3_ref, b3_ref, o_ref):
    z = a_ref[...] @ wo_ref[...] + bo_ref[...]
    z = jnp.maximum(z @ w2_ref[...] + b2_ref[...], 0.0)
    o_ref[...] = z @ w3_ref[...] + b3_ref[...]


def _final(att, Wo, bo, W2, b2, W3, b3):
    return pl.pallas_call(
        _final_body,
        grid=(NB,),
        in_specs=[
            pl.BlockSpec((BLK, DM), lambda i: (i, 0)),
            pl.BlockSpec((DM, DM), lambda i: (0, 0)),
            pl.BlockSpec((1, DM), lambda i: (0, 0)),
            pl.BlockSpec((DM, H), lambda i: (0, 0)),
            pl.BlockSpec((1, H), lambda i: (0, 0)),
            pl.BlockSpec((H, NCLASS), lambda i: (0, 0)),
            pl.BlockSpec((1, NCLASS), lambda i: (0, 0)),
        ],
        out_specs=pl.BlockSpec((BLK, NCLASS), lambda i: (i, 0)),
        out_shape=jax.ShapeDtypeStruct((NP, NCLASS), jnp.float32),
    )(att, Wo, bo.reshape(1, DM), W2, b2.reshape(1, H),
      W3, b3.reshape(1, NCLASS))


def kernel(x, edge_index, edge_weight, W0, b0, W1, b1, Wq, bq, Wk, bk,
           Wv, bv, Wo, bo, W2, b2, W3, b3):
    hcat = _proj(x, W0, b0, W1, b1)

    pad = EPAD - E
    srcr = jnp.pad(edge_index[0], (0, pad)).reshape(NS, NCH, CHUNK)
    dstr = jnp.pad(edge_index[1], (0, pad)).reshape(NS, NCH, CHUNK)
    wr = jnp.pad(edge_weight, (0, pad)).reshape(NS, NCH, CHUNK)

    out1, out2 = _sc_spmm(hcat.reshape(NC * NPS, H), srcr, dstr, wr)
    o1 = out1.reshape(NC, NPS, H)
    o2 = out2.reshape(NC, NPS, H)

    q, k, v = _qkv(o1, o2, Wq, bq, Wk, bk, Wv, bv)
    att = _flash(q, k, v)
    out = _final(att, Wo, bo, W2, b2, W3, b3)
    return out[:N]
